# baseline (speedup 1.0000x reference)
import functools

import jax
import jax.numpy as jnp
from jax import lax
from jax.experimental import pallas as pl
from jax.experimental.pallas import tpu as pltpu
from jax.experimental.pallas import tpu_sc as plsc

MAXLEN = 200
EMBED_DIM = 64
BATCH = 1024
VOCAB = 1000000

NUM_WORKERS = 32
PAIR_ROWS = VOCAB // 2
PAIR_W = 2 * EMBED_DIM

NGRP = VOCAB // 128
GRP_TAIL = VOCAB - NGRP * 128
G_PER_W = NGRP // NUM_WORKERS
G_REM = NGRP - G_PER_W * NUM_WORKERS

BBLK = 128
NBB = BATCH // BBLK
UNITS = MAXLEN * NBB
U_PER_W = UNITS // NUM_WORKERS
NCHUNK = BBLK // 16


def _transpose_block(in_blk, out_blk, ncols):
    c_iota = lax.iota(jnp.int32, 16)

    for q in range(ncols // 2):
        for j in range(PAIR_W // 16):
            half = j // (EMBED_DIM // 16)
            c0 = (j % (EMBED_DIM // 16)) * 16
            vr16 = jnp.full((16,), 2 * q + half, jnp.int32)
            val = plsc.load_gather(in_blk, [c_iota + c0, vr16])
            out_blk[q, pl.ds(j * 16, 16)] = val


def _tr_body(tokT_hbm, out_hbm, in_blk, out_blk, sem_i, sem_o):
    wid = lax.axis_index("s") * 2 + lax.axis_index("c")
    g0 = wid * G_PER_W

    def in_copy(g, b):
        return pltpu.make_async_copy(
            tokT_hbm.at[:, pl.ds(g * 128, 128)], in_blk.at[b], sem_i)

    def out_copy(g, b):
        return pltpu.make_async_copy(
            out_blk.at[b], out_hbm.at[pl.ds(g * 64, 64), :], sem_o)

    in_copy(g0, 0).start()

    def step(i, carry):
        for b in range(2):
            g = g0 + i * 2 + b

            @pl.when(g + 1 < g0 + G_PER_W)
            def _():
                in_copy(g + 1, 1 - b).start()

            in_copy(g, b).wait()
            _transpose_block(in_blk.at[b], out_blk.at[b], 128)

            @pl.when(i * 2 + b >= 2)
            def _():
                out_copy(g - 2, b).wait()

            out_copy(g, b).start()
        return carry

    lax.fori_loop(0, G_PER_W // 2, step, 0)
    out_copy(g0 + G_PER_W - 2, 0).wait()
    out_copy(g0 + G_PER_W - 1, 1).wait()

    @pl.when(wid < G_REM)
    def _():
        g = NUM_WORKERS * G_PER_W + wid
        in_copy(g, 0).start()
        in_copy(g, 0).wait()
        _transpose_block(in_blk.at[0], out_blk.at[0], 128)
        out_copy(g, 0).start()
        out_copy(g, 0).wait()

    @pl.when(wid == G_REM)
    def _():
        v0 = NGRP * 128
        for c in range(EMBED_DIM):
            pltpu.make_async_copy(
                tokT_hbm.at[c, pl.ds(v0, GRP_TAIL)],
                in_blk.at[0, c, pl.ds(0, GRP_TAIL)], sem_i).start()
        for c in range(EMBED_DIM):
            pltpu.make_async_copy(
                tokT_hbm.at[c, pl.ds(v0, GRP_TAIL)],
                in_blk.at[0, c, pl.ds(0, GRP_TAIL)], sem_i).wait()
        _transpose_block(in_blk.at[0], out_blk.at[0], GRP_TAIL)
        nt = GRP_TAIL // 2
        pltpu.make_async_copy(
            out_blk.at[0, pl.ds(0, nt), :],
            out_hbm.at[pl.ds(PAIR_ROWS - nt, nt), :], sem_o).start()
        pltpu.make_async_copy(
            out_blk.at[0, pl.ds(0, nt), :],
            out_hbm.at[pl.ds(PAIR_ROWS - nt, nt), :], sem_o).wait()


def _gx_body(idxT_hbm, tok2_hbm, pos_hbm, out_hbm,
             idx_raw, idx_g, base_v, pairs_v, slab_v, pos_v, sem_x, sem_g,
             sem_s):
    wid = lax.axis_index("s") * 2 + lax.axis_index("c")
    u0 = wid * U_PER_W

    pltpu.sync_copy(pos_hbm, pos_v)

    def idx_copy(u, b):
        p = u // NBB
        bb = u % NBB
        return pltpu.make_async_copy(
            idxT_hbm.at[p, pl.ds(bb * BBLK, BBLK)], idx_raw.at[b], sem_x)

    def prep_and_gather(u, b):
        for k in range(NCHUNK):
            sl = pl.ds(k * 16, 16)
            v16 = idx_raw[b, sl]
            idx_g[b, sl] = lax.shift_right_logical(v16, 1)
            base_v[b, sl] = lax.shift_left(jnp.bitwise_and(v16, 1), 6)
        pltpu.make_async_copy(
            tok2_hbm.at[idx_g.at[b]], pairs_v.at[b], sem_g).start()

    def gather_wait(b):
        pltpu.make_async_copy(
            tok2_hbm.at[idx_g.at[b]], pairs_v.at[b], sem_g).wait()

    def slab_copy(u, b):
        p = u // NBB
        bb = u % NBB
        return pltpu.make_async_copy(
            slab_v.at[b], out_hbm.at[p, :, pl.ds(bb * BBLK, BBLK)], sem_s)

    idx_copy(u0, 0).start()
    idx_copy(u0, 0).wait()
    prep_and_gather(u0, 0)
    idx_copy(u0 + 1, 1).start()

    def step(i, carry):
        for b in range(2):
            u = u0 + i * 2 + b

            @pl.when(u + 1 < u0 + U_PER_W)
            def _():
                idx_copy(u + 1, 1 - b).wait()
                prep_and_gather(u + 1, 1 - b)

            @pl.when(u + 2 < u0 + U_PER_W)
            def _():
                idx_copy(u + 2, b).start()

            gather_wait(b)

            @pl.when(i * 2 + b >= 2)
            def _():
                slab_copy(u - 2, b).wait()

            p = u // NBB

            for k in range(NCHUNK):
                sl = pl.ds(k * 16, 16)
                half16 = base_v[b, sl]
                row16 = lax.iota(jnp.int32, 16) + (k * 16)
                for cc in range(EMBED_DIM // 16):
                    pos16 = pos_v[p, pl.ds(cc * 16, 16)]
                    for cl in range(16):
                        c = cc * 16 + cl
                        val = plsc.load_gather(
                            pairs_v.at[b], [row16, half16 + c])
                        slab_v[b, c, sl] = val + pos16[cl]

            slab_copy(u, b).start()
        return carry

    lax.fori_loop(0, U_PER_W // 2, step, 0)
    slab_copy(u0 + U_PER_W - 2, 0).wait()
    slab_copy(u0 + U_PER_W - 1, 1).wait()


def kernel(inputs, token_table, pos_table):
    idxT = inputs.T.astype(jnp.int32)
    tokT = token_table.T
    mesh = plsc.VectorSubcoreMesh(core_axis_name="c", subcore_axis_name="s")
    params = pltpu.CompilerParams(
        use_tc_tiling_on_sc=True, needs_layout_passes=False)

    tr = functools.partial(
        pl.kernel,
        out_type=jax.ShapeDtypeStruct((PAIR_ROWS, PAIR_W), jnp.float32),
        mesh=mesh,
        scratch_types=[
            pltpu.VMEM((2, EMBED_DIM, 128), jnp.float32),
            pltpu.VMEM((2, EMBED_DIM, 128), jnp.float32),
            pltpu.SemaphoreType.DMA,
            pltpu.SemaphoreType.DMA,
        ],
        compiler_params=params,
    )(_tr_body)
    tok2 = tr(tokT)

    gx = functools.partial(
        pl.kernel,
        out_type=jax.ShapeDtypeStruct((MAXLEN, EMBED_DIM, BATCH), jnp.float32),
        mesh=mesh,
        scratch_types=[
            pltpu.VMEM((2, BBLK), jnp.int32),
            pltpu.VMEM((2, BBLK), jnp.int32),
            pltpu.VMEM((2, BBLK), jnp.int32),
            pltpu.VMEM((2, BBLK, PAIR_W), jnp.float32),
            pltpu.VMEM((2, EMBED_DIM, BBLK), jnp.float32),
            pltpu.VMEM((MAXLEN, EMBED_DIM), jnp.float32),
            pltpu.SemaphoreType.DMA,
            pltpu.SemaphoreType.DMA,
            pltpu.SemaphoreType.DMA,
        ],
        compiler_params=params,
    )(_gx_body)
    out = gx(idxT, tok2, pos_table)
    return jnp.transpose(out, (2, 0, 1))


# speedup vs baseline: 2.5630x; 2.5630x over previous
import functools

import jax
import jax.numpy as jnp
from jax import lax
from jax.experimental import pallas as pl
from jax.experimental.pallas import tpu as pltpu
from jax.experimental.pallas import tpu_sc as plsc

MAXLEN = 200
EMBED_DIM = 64
BATCH = 1024

NUM_WORKERS = 32
B_PER_W = BATCH // NUM_WORKERS
IDX_SPLIT = 2
IDX_MINOR = MAXLEN // IDX_SPLIT
VECS_PER_ROW = EMBED_DIM // 16
NBUF = 4
GATHER_AHEAD = 2


def _sc_body(idx_hbm, tok_hbm, pos_hbm, out_hbm, idx_v, rows_v, pos_v,
             sem_g, sem_o):
    wid = lax.axis_index("s") * 2 + lax.axis_index("c")
    b0 = wid * B_PER_W

    pltpu.sync_copy(pos_hbm, pos_v)
    pltpu.sync_copy(idx_hbm.at[pl.ds(b0, B_PER_W)], idx_v)

    def start_gather(i, b):
        for c in range(IDX_SPLIT):
            pltpu.make_async_copy(
                tok_hbm.at[idx_v.at[i, c]],
                rows_v.at[b, pl.ds(c * IDX_MINOR, IDX_MINOR)],
                sem_g,
            ).start()

    def wait_gather(i, b):
        for c in range(IDX_SPLIT):
            pltpu.make_async_copy(
                tok_hbm.at[idx_v.at[i, c]],
                rows_v.at[b, pl.ds(c * IDX_MINOR, IDX_MINOR)],
                sem_g,
            ).wait()

    def out_copy(i, b):
        return pltpu.make_async_copy(rows_v.at[b], out_hbm.at[b0 + i], sem_o)

    for i in range(GATHER_AHEAD):
        start_gather(i, i % NBUF)

    def outer(io, carry):
        for b_off in range(NBUF):
            i = io * NBUF + b_off
            b = b_off
            @pl.when(i >= NBUF - GATHER_AHEAD)
            def _():
                out_copy(i - (NBUF - GATHER_AHEAD),
                         (i + GATHER_AHEAD) % NBUF).wait()

            @pl.when(i + GATHER_AHEAD < B_PER_W)
            def _():
                start_gather(i + GATHER_AHEAD, (i + GATHER_AHEAD) % NBUF)

            wait_gather(i, b)

            @plsc.parallel_loop(0, MAXLEN, 1, unroll=4)
            def add_row(r):
                for j in range(VECS_PER_ROW):
                    sl = pl.ds(j * 16, 16)
                    rows_v[b, r, sl] = rows_v[b, r, sl] + pos_v[r, sl]

            out_copy(i, b).start()
        return carry

    lax.fori_loop(0, B_PER_W // NBUF, outer, 0)

    for i in range(B_PER_W - (NBUF - GATHER_AHEAD), B_PER_W):
        out_copy(i, i % NBUF).wait()


def kernel(inputs, token_table, pos_table):
    idx3 = inputs.reshape(BATCH, IDX_SPLIT, IDX_MINOR).astype(jnp.int32)
    mesh = plsc.VectorSubcoreMesh(core_axis_name="c", subcore_axis_name="s")
    k = functools.partial(
        pl.kernel,
        out_type=jax.ShapeDtypeStruct((BATCH, MAXLEN, EMBED_DIM), jnp.float32),
        mesh=mesh,
        scratch_types=[
            pltpu.VMEM((B_PER_W, IDX_SPLIT, IDX_MINOR), jnp.int32),
            pltpu.VMEM((NBUF, MAXLEN, EMBED_DIM), jnp.float32),
            pltpu.VMEM((MAXLEN, EMBED_DIM), jnp.float32),
            pltpu.SemaphoreType.DMA,
            pltpu.SemaphoreType.DMA,
        ],
        compiler_params=pltpu.CompilerParams(use_tc_tiling_on_sc=False),
    )(_sc_body)
    return k(idx3, token_table, pos_table)
